# baseline (speedup 1.0000x reference)
import jax
import jax.numpy as jnp
from jax import lax
from jax.experimental import pallas as pl
from jax.experimental.pallas import tpu as pltpu
from jax.experimental.pallas import tpu_sc as plsc

B, N, E, D = 4, 12500, 200000, 64

NC, NS = 2, 16
NPAD = 12544
RPT = NPAD // NS
EPT = E // NS
CH = 512
NCHUNK = 25
EPTP = NCHUNK * CH
BPC = B // NC

ROWS = B * NPAD
BLK = 512
GRID = ROWS // BLK

import functools


@functools.cache
def _mesh():
    return plsc.VectorSubcoreMesh(
        core_axis_name="c", subcore_axis_name="s", num_cores=NC, num_subcores=NS
    )


def _sc_deg_body(rows_hbm, ones_hbm, zeros_hbm, deg_hbm, acc_sh, idx_v, ones_v):
    c = lax.axis_index("c")
    s = lax.axis_index("s")
    pltpu.sync_copy(ones_hbm, ones_v)
    for i in range(BPC):
        b = c * BPC + i
        pltpu.sync_copy(zeros_hbm, acc_sh.at[pl.ds(s * RPT, RPT)])
        pltpu.sync_copy(rows_hbm.at[b, s], idx_v)
        plsc.subcore_barrier()

        def chunk(j, carry):
            pltpu.sync_copy(ones_v, acc_sh.at[idx_v.at[j]], add=True)
            return carry

        lax.fori_loop(0, NCHUNK, chunk, 0)
        plsc.subcore_barrier()
        pltpu.sync_copy(acc_sh.at[pl.ds(s * RPT, RPT)],
                        deg_hbm.at[b, pl.ds(s * RPT, RPT)])
        plsc.subcore_barrier()


@functools.cache
def _sc_deg():
    return pl.kernel(
        _sc_deg_body,
        out_type=jax.ShapeDtypeStruct((B, NPAD, D), jnp.float32),
        mesh=_mesh(),
        scratch_types=[
            pltpu.VMEM_SHARED((NPAD, D), jnp.float32),
            pltpu.VMEM((NCHUNK, CH), jnp.int32),
            pltpu.VMEM((CH, D), jnp.float32),
        ],
        compiler_params=pltpu.CompilerParams(use_tc_tiling_on_sc=False),
    )


def _sc_prop_body(g_hbm, rows_hbm, cols_hbm, zeros_hbm, t_hbm,
                  acc_sh, rid_v, cid_v, buf_v):
    c = lax.axis_index("c")
    s = lax.axis_index("s")
    for i in range(BPC):
        b = c * BPC + i
        pltpu.sync_copy(zeros_hbm, acc_sh.at[pl.ds(s * RPT, RPT)])
        pltpu.sync_copy(rows_hbm.at[b, s], rid_v)
        pltpu.sync_copy(cols_hbm.at[b, s], cid_v)
        plsc.subcore_barrier()

        def chunk(j, carry):
            pltpu.sync_copy(g_hbm.at[cid_v.at[j]], buf_v)
            pltpu.sync_copy(buf_v, acc_sh.at[rid_v.at[j]], add=True)
            return carry

        lax.fori_loop(0, NCHUNK, chunk, 0)
        plsc.subcore_barrier()
        pltpu.sync_copy(acc_sh.at[pl.ds(s * RPT, RPT)],
                        t_hbm.at[b, pl.ds(s * RPT, RPT)])
        plsc.subcore_barrier()


@functools.cache
def _sc_prop():
    return pl.kernel(
        _sc_prop_body,
        out_type=jax.ShapeDtypeStruct((B, NPAD, D), jnp.float32),
        mesh=_mesh(),
        scratch_types=[
            pltpu.VMEM_SHARED((NPAD, D), jnp.float32),
            pltpu.VMEM((NCHUNK, CH), jnp.int32),
            pltpu.VMEM((NCHUNK, CH), jnp.int32),
            pltpu.VMEM((CH, D), jnp.float32),
        ],
        compiler_params=pltpu.CompilerParams(use_tc_tiling_on_sc=False),
    )


def _tc_first_body(x_ref, deg_ref, w_ref, b_ref, g_ref, dis_ref):
    deg = deg_ref[...]
    dis = jnp.where(deg > 0.0, lax.rsqrt(jnp.maximum(deg, 1.0)), 0.0)
    dis_ref[...] = dis
    z = jnp.dot(x_ref[...], w_ref[...], preferred_element_type=jnp.float32)
    g_ref[...] = dis * (z + b_ref[...])


def _tc_mid_body(t_ref, dis_ref, w_ref, b_ref, g_ref):
    dis = dis_ref[...]
    h = jnp.maximum(dis * t_ref[...], 0.0)
    z = jnp.dot(h, w_ref[...], preferred_element_type=jnp.float32)
    g_ref[...] = dis * (z + b_ref[...])


def _tc_out_body(t_ref, dis_ref, o_ref):
    o_ref[...] = dis_ref[...] * t_ref[...]


_row_spec = pl.BlockSpec((BLK, D), lambda i: (i, 0))
_w_spec = pl.BlockSpec((D, D), lambda i: (0, 0))
_b_spec = pl.BlockSpec((1, D), lambda i: (0, 0))

_tc_first = pl.pallas_call(
    _tc_first_body,
    grid=(GRID,),
    in_specs=[_row_spec, _row_spec, _w_spec, _b_spec],
    out_specs=[_row_spec, _row_spec],
    out_shape=[
        jax.ShapeDtypeStruct((ROWS, D), jnp.float32),
        jax.ShapeDtypeStruct((ROWS, D), jnp.float32),
    ],
)

_tc_mid = pl.pallas_call(
    _tc_mid_body,
    grid=(GRID,),
    in_specs=[_row_spec, _row_spec, _w_spec, _b_spec],
    out_specs=_row_spec,
    out_shape=jax.ShapeDtypeStruct((ROWS, D), jnp.float32),
)

_tc_out = pl.pallas_call(
    _tc_out_body,
    grid=(GRID,),
    in_specs=[_row_spec, _row_spec],
    out_specs=_row_spec,
    out_shape=jax.ShapeDtypeStruct((ROWS, D), jnp.float32),
)


def kernel(x, edge_index, W1, b1, W2, b2, W3, b3):
    row = edge_index[:, :, 0].reshape(B, NS, EPT)
    col = edge_index[:, :, 1].reshape(B, NS, EPT)
    pad = EPTP - EPT
    row_c = jnp.pad(row, ((0, 0), (0, 0), (0, pad)), constant_values=N)
    col_c = jnp.pad(col, ((0, 0), (0, 0), (0, pad)), constant_values=0)
    row_c = row_c.reshape(B, NS, NCHUNK, CH)
    col_c = (col_c + jnp.arange(B, dtype=jnp.int32)[:, None, None] * NPAD)
    col_c = col_c.reshape(B, NS, NCHUNK, CH)

    x_p = jnp.pad(x, ((0, 0), (0, NPAD - N), (0, 0)))
    zeros = jnp.zeros((RPT, D), jnp.float32)
    ones = jnp.ones((CH, D), jnp.float32)
    b1r, b2r, b3r = (b.reshape(1, D) for b in (b1, b2, b3))

    sc_deg, sc_prop = _sc_deg(), _sc_prop()
    deg = sc_deg(row_c, ones, zeros)
    g1, dis = _tc_first(x_p.reshape(ROWS, D), deg.reshape(ROWS, D), W1.T, b1r)
    t1 = sc_prop(g1, row_c, col_c, zeros)
    g2 = _tc_mid(t1.reshape(ROWS, D), dis, W2.T, b2r)
    t2 = sc_prop(g2, row_c, col_c, zeros)
    g3 = _tc_mid(t2.reshape(ROWS, D), dis, W3.T, b3r)
    t3 = sc_prop(g3, row_c, col_c, zeros)
    out = _tc_out(t3.reshape(ROWS, D), dis)
    return out.reshape(B, NPAD, D)[:, :N, :]


# speedup vs baseline: 1.1863x; 1.1863x over previous
import jax
import jax.numpy as jnp
from jax import lax
from jax.experimental import pallas as pl
from jax.experimental.pallas import tpu as pltpu
from jax.experimental.pallas import tpu_sc as plsc

B, N, E, D = 4, 12500, 200000, 64

NC, NS = 2, 16
NPAD = 12544
RPT = NPAD // NS
EPT = E // NS
CH = 128
NCHUNK = 98
EPTP = NCHUNK * CH
WIN = 14
NWIN = NCHUNK // WIN
BPC = B // NC

ROWS = B * NPAD
BLK = 512
GRID = ROWS // BLK

import functools


@functools.cache
def _mesh():
    return plsc.VectorSubcoreMesh(
        core_axis_name="c", subcore_axis_name="s", num_cores=NC, num_subcores=NS
    )


def _sc_deg_body(rows_hbm, ones_hbm, zeros_hbm, deg_hbm, acc_sh, idx_v, ones_v):
    c = lax.axis_index("c")
    s = lax.axis_index("s")
    pltpu.sync_copy(ones_hbm, ones_v)
    for i in range(BPC):
        b = c * BPC + i
        pltpu.sync_copy(zeros_hbm, acc_sh.at[pl.ds(s * RPT, RPT)])
        pltpu.sync_copy(rows_hbm.at[b, s], idx_v)
        plsc.subcore_barrier()

        def chunk(j, carry):
            pltpu.sync_copy(ones_v, acc_sh.at[idx_v.at[j]], add=True)
            return carry

        lax.fori_loop(0, NCHUNK, chunk, 0)
        plsc.subcore_barrier()
        pltpu.sync_copy(acc_sh.at[pl.ds(s * RPT, RPT)],
                        deg_hbm.at[b, pl.ds(s * RPT, RPT)])
        plsc.subcore_barrier()


@functools.cache
def _sc_deg():
    return pl.kernel(
        _sc_deg_body,
        out_type=jax.ShapeDtypeStruct((B, NPAD, D), jnp.float32),
        mesh=_mesh(),
        scratch_types=[
            pltpu.VMEM_SHARED((NPAD, D), jnp.float32),
            pltpu.VMEM((NCHUNK, CH), jnp.int32),
            pltpu.VMEM((CH, D), jnp.float32),
        ],
        compiler_params=pltpu.CompilerParams(use_tc_tiling_on_sc=False),
    )


def _sc_prop_body(g_hbm, rows_hbm, cols_hbm, zeros_hbm, t_hbm,
                  g_sh, acc_sh, rid_v, cid_v, buf_v):
    c = lax.axis_index("c")
    s = lax.axis_index("s")
    for i in range(BPC):
        b = c * BPC + i
        pltpu.sync_copy(g_hbm.at[b, pl.ds(s * RPT, RPT)],
                        g_sh.at[pl.ds(s * RPT, RPT)])
        pltpu.sync_copy(zeros_hbm, acc_sh.at[pl.ds(s * RPT, RPT)])
        plsc.subcore_barrier()

        def window(w, carry):
            pltpu.sync_copy(rows_hbm.at[b, s, pl.ds(w * WIN, WIN)], rid_v)
            pltpu.sync_copy(cols_hbm.at[b, s, pl.ds(w * WIN, WIN)], cid_v)

            def chunk(j, carry2):
                pltpu.sync_copy(g_sh.at[cid_v.at[j]], buf_v)
                pltpu.sync_copy(buf_v, acc_sh.at[rid_v.at[j]], add=True)
                return carry2

            lax.fori_loop(0, WIN, chunk, 0)
            return carry

        lax.fori_loop(0, NWIN, window, 0)
        plsc.subcore_barrier()
        pltpu.sync_copy(acc_sh.at[pl.ds(s * RPT, RPT)],
                        t_hbm.at[b, pl.ds(s * RPT, RPT)])
        plsc.subcore_barrier()


@functools.cache
def _sc_prop():
    return pl.kernel(
        _sc_prop_body,
        out_type=jax.ShapeDtypeStruct((B, NPAD, D), jnp.float32),
        mesh=_mesh(),
        scratch_types=[
            pltpu.VMEM_SHARED((NPAD, D), jnp.float32),
            pltpu.VMEM_SHARED((NPAD, D), jnp.float32),
            pltpu.VMEM((WIN, CH), jnp.int32),
            pltpu.VMEM((WIN, CH), jnp.int32),
            pltpu.VMEM((CH, D), jnp.float32),
        ],
        compiler_params=pltpu.CompilerParams(use_tc_tiling_on_sc=False),
    )


def _tc_first_body(x_ref, deg_ref, w_ref, b_ref, g_ref, dis_ref):
    deg = deg_ref[...]
    dis = jnp.where(deg > 0.0, lax.rsqrt(jnp.maximum(deg, 1.0)), 0.0)
    dis_ref[...] = dis
    z = jnp.dot(x_ref[...], w_ref[...], preferred_element_type=jnp.float32)
    g_ref[...] = dis * (z + b_ref[...])


def _tc_mid_body(t_ref, dis_ref, w_ref, b_ref, g_ref):
    dis = dis_ref[...]
    h = jnp.maximum(dis * t_ref[...], 0.0)
    z = jnp.dot(h, w_ref[...], preferred_element_type=jnp.float32)
    g_ref[...] = dis * (z + b_ref[...])


def _tc_out_body(t_ref, dis_ref, o_ref):
    o_ref[...] = dis_ref[...] * t_ref[...]


_row_spec = pl.BlockSpec((BLK, D), lambda i: (i, 0))
_w_spec = pl.BlockSpec((D, D), lambda i: (0, 0))
_b_spec = pl.BlockSpec((1, D), lambda i: (0, 0))

_tc_first = pl.pallas_call(
    _tc_first_body,
    grid=(GRID,),
    in_specs=[_row_spec, _row_spec, _w_spec, _b_spec],
    out_specs=[_row_spec, _row_spec],
    out_shape=[
        jax.ShapeDtypeStruct((ROWS, D), jnp.float32),
        jax.ShapeDtypeStruct((ROWS, D), jnp.float32),
    ],
)

_tc_mid = pl.pallas_call(
    _tc_mid_body,
    grid=(GRID,),
    in_specs=[_row_spec, _row_spec, _w_spec, _b_spec],
    out_specs=_row_spec,
    out_shape=jax.ShapeDtypeStruct((ROWS, D), jnp.float32),
)

_tc_out = pl.pallas_call(
    _tc_out_body,
    grid=(GRID,),
    in_specs=[_row_spec, _row_spec],
    out_specs=_row_spec,
    out_shape=jax.ShapeDtypeStruct((ROWS, D), jnp.float32),
)


def kernel(x, edge_index, W1, b1, W2, b2, W3, b3):
    row = edge_index[:, :, 0].reshape(B, NS, EPT)
    col = edge_index[:, :, 1].reshape(B, NS, EPT)
    pad = EPTP - EPT
    row_c = jnp.pad(row, ((0, 0), (0, 0), (0, pad)), constant_values=N)
    col_c = jnp.pad(col, ((0, 0), (0, 0), (0, pad)), constant_values=0)
    row_c = row_c.reshape(B, NS, NCHUNK, CH)
    col_c = col_c.reshape(B, NS, NCHUNK, CH)

    x_p = jnp.pad(x, ((0, 0), (0, NPAD - N), (0, 0)))
    zeros = jnp.zeros((RPT, D), jnp.float32)
    ones = jnp.ones((CH, D), jnp.float32)
    b1r, b2r, b3r = (b.reshape(1, D) for b in (b1, b2, b3))

    sc_deg, sc_prop = _sc_deg(), _sc_prop()
    deg = sc_deg(row_c, ones, zeros)
    g1, dis = _tc_first(x_p.reshape(ROWS, D), deg.reshape(ROWS, D), W1.T, b1r)
    t1 = sc_prop(g1.reshape(B, NPAD, D), row_c, col_c, zeros)
    g2 = _tc_mid(t1.reshape(ROWS, D), dis, W2.T, b2r)
    t2 = sc_prop(g2.reshape(B, NPAD, D), row_c, col_c, zeros)
    g3 = _tc_mid(t2.reshape(ROWS, D), dis, W3.T, b3r)
    t3 = sc_prop(g3.reshape(B, NPAD, D), row_c, col_c, zeros)
    out = _tc_out(t3.reshape(ROWS, D), dis)
    return out.reshape(B, NPAD, D)[:, :N, :]
